# Initial kernel scaffold; baseline (speedup 1.0000x reference)
import jax
import jax.numpy as jnp
from jax.experimental import pallas as pl


def kernel(edge_index, entity_embeddings, W1, b1, W2, b2, Wc, bc):
    raise NotImplementedError("write your pallas kernel here")


# speedup vs baseline: 5.1497x; 5.1497x over previous
import functools

import jax
import jax.numpy as jnp
from jax import lax
from jax.experimental import pallas as pl
from jax.experimental.pallas import tpu as pltpu
from jax.experimental.pallas import tpu_sc as plsc

EPS = 1e-15
_CLIP = 1.0 - 1e-6


def _logmap0(x):
    norm = jnp.maximum(jnp.sqrt(jnp.sum(x * x, axis=-1, keepdims=True)), EPS)
    arg = jnp.clip(norm, 0.0, _CLIP)
    atanh = 0.5 * jnp.log((1.0 + arg) / (1.0 - arg))
    return x * atanh / norm


def _expmap0(u):
    norm = jnp.maximum(jnp.sqrt(jnp.sum(u * u, axis=-1, keepdims=True)), EPS)
    return jnp.tanh(norm) * u / norm


def _dense_layer_body(x_ref, w_ref, b_ref, o_ref):
    x = x_ref[...]
    t = _logmap0(x)
    h = lax.dot_general(t, w_ref[...], (((1,), (1,)), ((), ())),
                        preferred_element_type=jnp.float32) + b_ref[...]
    o_ref[...] = _expmap0(h)


def _dense_layer_mid_body(p0_ref, p1_ref, w_ref, b_ref, o_ref):
    x = jnp.maximum(p0_ref[0] + p1_ref[0], 0.0)
    t = _logmap0(x)
    h = lax.dot_general(t, w_ref[...], (((1,), (1,)), ((), ())),
                        preferred_element_type=jnp.float32) + b_ref[...]
    o_ref[...] = _expmap0(h)


def _classifier_body(p0_ref, p1_ref, w_ref, b_ref, o_ref):
    x = jnp.maximum(p0_ref[0] + p1_ref[0], 0.0)
    t = _logmap0(x)
    o_ref[...] = lax.dot_general(t, w_ref[...], (((1,), (1,)), ((), ())),
                                 preferred_element_type=jnp.float32) + b_ref[...]


def _dense_first(x, W, b):
    n, d = x.shape
    blk = 2000
    grid = n // blk
    return pl.pallas_call(
        _dense_layer_body,
        grid=(grid,),
        in_specs=[
            pl.BlockSpec((blk, d), lambda i: (i, 0)),
            pl.BlockSpec((d, d), lambda i: (0, 0)),
            pl.BlockSpec((1, d), lambda i: (0, 0)),
        ],
        out_specs=pl.BlockSpec((blk, d), lambda i: (i, 0)),
        out_shape=jax.ShapeDtypeStruct((n, d), jnp.float32),
    )(x, W, b.reshape(1, d))


def _dense_mid(partials, W, b, n):
    d = partials.shape[2]
    blk = 2000
    grid = n // blk
    return pl.pallas_call(
        _dense_layer_mid_body,
        grid=(grid,),
        in_specs=[
            pl.BlockSpec((1, blk, d), lambda i: (0, i, 0)),
            pl.BlockSpec((1, blk, d), lambda i: (1, i, 0)),
            pl.BlockSpec((d, d), lambda i: (0, 0)),
            pl.BlockSpec((1, d), lambda i: (0, 0)),
        ],
        out_specs=pl.BlockSpec((blk, d), lambda i: (i, 0)),
        out_shape=jax.ShapeDtypeStruct((n, d), jnp.float32),
    )(partials, partials, W, b.reshape(1, d))


def _classifier(partials, Wc, bc, n):
    d = partials.shape[2]
    nc = Wc.shape[0]
    ncp = 16
    Wp = jnp.zeros((ncp, d), jnp.float32).at[:nc].set(Wc)
    bp = jnp.zeros((ncp,), jnp.float32).at[:nc].set(bc)
    blk = 2000
    grid = n // blk
    out = pl.pallas_call(
        _classifier_body,
        grid=(grid,),
        in_specs=[
            pl.BlockSpec((1, blk, d), lambda i: (0, i, 0)),
            pl.BlockSpec((1, blk, d), lambda i: (1, i, 0)),
            pl.BlockSpec((ncp, d), lambda i: (0, 0)),
            pl.BlockSpec((1, ncp), lambda i: (0, 0)),
        ],
        out_specs=pl.BlockSpec((blk, ncp), lambda i: (i, 0)),
        out_shape=jax.ShapeDtypeStruct((n, ncp), jnp.float32),
    )(partials, partials, Wp, bp.reshape(1, ncp))
    return out[:, :nc]


def _scatter_partials(y, src, dst):
    n, d = y.shape
    e = src.shape[0]
    NC, NS = 2, 16
    NW = NC * NS
    epw = e // NW
    K = 80
    steps = epw // K
    NP = 10240
    rpt = NP // NS
    ZR = 128

    mesh = plsc.VectorSubcoreMesh(core_axis_name="c", subcore_axis_name="s")

    @functools.partial(
        pl.kernel,
        mesh=mesh,
        out_type=jax.ShapeDtypeStruct((NC, NP, d), jnp.float32),
        scratch_types=[
            pltpu.VMEM((K,), jnp.int32),
            pltpu.VMEM((K,), jnp.int32),
            pltpu.VMEM((K, d), jnp.float32),
            pltpu.VMEM((ZR, d), jnp.float32),
            pltpu.VMEM_SHARED((NP, d), jnp.float32),
            pltpu.SemaphoreType.DMA,
        ],
    )
    def k(src_hbm, dst_hbm, y_hbm, out_hbm, src_v, dst_v, rows_v, zbuf, acc, sem):
        c = lax.axis_index("c")
        s = lax.axis_index("s")
        wid = c * NS + s

        def zrow(i, carry):
            for j in range(d // 16):
                zbuf[i, pl.ds(j * 16, 16)] = jnp.zeros((16,), jnp.float32)
            return carry

        lax.fori_loop(0, ZR, zrow, 0)
        for r in range(rpt // ZR):
            pltpu.sync_copy(zbuf, acc.at[pl.ds(s * rpt + r * ZR, ZR)])
        plsc.subcore_barrier()

        base0 = wid * epw

        def step(i, carry):
            base = base0 + i * K
            pltpu.sync_copy(src_hbm.at[pl.ds(base, K)], src_v)
            pltpu.sync_copy(dst_hbm.at[pl.ds(base, K)], dst_v)
            pltpu.async_copy(y_hbm.at[src_v], rows_v, sem).wait()
            pltpu.sync_copy(rows_v, acc.at[dst_v], add=True)
            return carry

        lax.fori_loop(0, steps, step, 0)
        plsc.subcore_barrier()
        pltpu.sync_copy(acc.at[pl.ds(s * rpt, rpt)],
                        out_hbm.at[c, pl.ds(s * rpt, rpt)])

    return k(src, dst, y)


def kernel(edge_index, entity_embeddings, W1, b1, W2, b2, Wc, bc):
    src = edge_index[0]
    dst = edge_index[1]
    n = entity_embeddings.shape[0]
    y1 = _dense_first(entity_embeddings, W1, b1)
    p1 = _scatter_partials(y1, src, dst)
    y2 = _dense_mid(p1, W2, b2, n)
    p2 = _scatter_partials(y2, src, dst)
    return _classifier(p2, Wc, bc, n)
